# baseline (speedup 1.0000x reference)
import functools

import jax
import jax.numpy as jnp
import numpy as np
from jax import lax
from jax.experimental import pallas as pl
from jax.experimental.pallas import tpu as pltpu
from jax.experimental.pallas import tpu_sc as plsc

N = 10000
E = 320000
C = 128

NC = 2
NS = 16
NW = NC * NS
EPW = E // NW
K = 40
NCHUNK = EPW // K
RZB = 640


def _dot_t(x, w):
    return lax.dot_general(x, w, (((1,), (1,)), ((), ())),
                           preferred_element_type=jnp.float32)


def _relu(x):
    return jnp.maximum(x, 0.0)


def _pre_math(h, posp, Wh1, bh1, Wh2p, bh2p, Wfpp, Wfx, bfr):
    z1 = _relu(_dot_t(h, Wh1) + bh1)
    d = jnp.tanh(_dot_t(z1, Wh2p) + bh2p)
    pP = _dot_t(posp, Wfpp)
    U = pP + _dot_t(h, Wfx)
    V = _dot_t(d, Wfpp) - pP + bfr
    return U, V


def _post_math(x, agg, style, Wg1, bg1, Wg2, bg2, Wsg, bsg, Wsb, bsb):
    g1 = _relu(_dot_t(agg, Wg1) + bg1)
    o = _relu(_dot_t(g1, Wg2) + bg2)
    t = x + o
    gam = _dot_t(style, Wsg) + bsg
    bet = _dot_t(style, Wsb) + bsb
    mu = jnp.mean(t, axis=1, keepdims=True)
    var = jnp.mean((t - mu) * (t - mu), axis=1, keepdims=True)
    y = gam * ((t - mu) * lax.rsqrt(var + 1e-5)) + bet
    return jnp.where(y >= 0, y, 0.01 * y)


RT = 1000
GRID = N // RT

_row = lambda r, c=C: pl.BlockSpec((RT, c), lambda i: (i, 0))
_full = lambda a, b: pl.BlockSpec((a, b), lambda i: (0, 0))
_vec = lambda c=C: pl.BlockSpec((1, c), lambda i: (0, 0))


def _tc_pre_body(h, posp, Wh1, bh1, Wh2p, bh2p, Wfpp, Wfx, bfr, U, V):
    u, v = _pre_math(h[...], posp[...], Wh1[...], bh1[...], Wh2p[...],
                     bh2p[...], Wfpp[...], Wfx[...], bfr[...])
    U[...] = u
    V[...] = v


def _tc_mid_body(x, agg2, style, posp,
                 Wg1, bg1, Wg2, bg2, Wsg, bsg, Wsb, bsb,
                 Wh1, bh1, Wh2p, bh2p, Wfpp, Wfx, bfr,
                 Y, U, V):
    agg = agg2[0] + agg2[1]
    y = _post_math(x[...], agg, style[...], Wg1[...], bg1[...], Wg2[...],
                   bg2[...], Wsg[...], bsg[...], Wsb[...], bsb[...])
    Y[...] = y
    u, v = _pre_math(y, posp[...], Wh1[...], bh1[...], Wh2p[...], bh2p[...],
                     Wfpp[...], Wfx[...], bfr[...])
    U[...] = u
    V[...] = v


def _tc_post_body(x, agg2, style,
                  Wg1, bg1, Wg2, bg2, Wsg, bsg, Wsb, bsb, Y):
    agg = agg2[0] + agg2[1]
    Y[...] = _post_math(x[...], agg, style[...], Wg1[...], bg1[...],
                        Wg2[...], bg2[...], Wsg[...], bsg[...], Wsb[...],
                        bsb[...])


_W_SPECS = [_full(C, C), _vec(), _full(C, C), _vec(), _full(C, C), _vec(),
            _full(C, C), _vec()]
_PRE_W_SPECS = [_full(C, C), _vec(), _full(C, C), _vec(), _full(C, C),
                _full(C, C), _vec()]

_tc_pre = pl.pallas_call(
    _tc_pre_body,
    grid=(GRID,),
    in_specs=[_row(RT), _row(RT)] + _PRE_W_SPECS,
    out_specs=[_row(RT), _row(RT)],
    out_shape=[jax.ShapeDtypeStruct((N, C), jnp.float32)] * 2,
)

_agg_spec = pl.BlockSpec((2, RT, C), lambda i: (0, i, 0))

_tc_mid = pl.pallas_call(
    _tc_mid_body,
    grid=(GRID,),
    in_specs=[_row(RT), _agg_spec, _row(RT), _row(RT)] + _W_SPECS + _PRE_W_SPECS,
    out_specs=[_row(RT), _row(RT), _row(RT)],
    out_shape=[jax.ShapeDtypeStruct((N, C), jnp.float32)] * 3,
)

_tc_post = pl.pallas_call(
    _tc_post_body,
    grid=(GRID,),
    in_specs=[_row(RT), _agg_spec, _row(RT)] + _W_SPECS,
    out_specs=_row(RT),
    out_shape=jax.ShapeDtypeStruct((N, C), jnp.float32),
)


KI = 8
PF = 6


def _sc_edges_body(u_hbm, v_hbm, src_hbm, dst_hbm, out_hbm, *scr):
    ur = scr[0:2]
    vr = scr[2:4]
    mr = scr[4:6]
    si_r = scr[6:6 + KI]
    di_r = scr[6 + KI:6 + 2 * KI]
    acc = scr[6 + 2 * KI]
    base_s = 7 + 2 * KI
    sem_u = scr[base_s:base_s + 2]
    sem_v = scr[base_s + 2:base_s + 4]
    sem_s = scr[base_s + 4:base_s + 6]
    sem_i = scr[base_s + 6:base_s + 6 + KI]

    c = lax.axis_index("c")
    s = lax.axis_index("s")
    wid = s * NC + c

    ebase = wid * EPW

    def issue_idx(ch, slot):
        pltpu.async_copy(src_hbm.at[pl.ds(ebase + ch * K, K)], si_r[slot],
                         sem_i[slot])
        pltpu.async_copy(dst_hbm.at[pl.ds(ebase + ch * K, K)], di_r[slot],
                         sem_i[slot])

    def wait_idx(slot):
        pltpu.make_async_copy(src_hbm.at[pl.ds(0, K)], si_r[slot],
                              sem_i[slot]).wait()
        pltpu.make_async_copy(dst_hbm.at[pl.ds(0, K)], di_r[slot],
                              sem_i[slot]).wait()

    def issue_gathers(b, slot):
        pltpu.async_copy(u_hbm.at[si_r[slot]], ur[b], sem_u[b])
        pltpu.async_copy(v_hbm.at[di_r[slot]], vr[b], sem_v[b])

    def wait_gathers(b, slot):
        pltpu.make_async_copy(u_hbm.at[si_r[slot]], ur[b], sem_u[b]).wait()
        pltpu.make_async_copy(v_hbm.at[di_r[slot]], vr[b], sem_v[b]).wait()

    for ch0 in range(PF):
        issue_idx(ch0, ch0)
    wait_idx(0)
    issue_gathers(0, 0)
    wait_idx(1)
    issue_gathers(1, 1)

    zv = jnp.zeros((16,), jnp.float32)

    def zero_row(i, _):
        for j in range(C // 16):
            mr[0][i, pl.ds(j * 16, 16)] = zv
            mr[1][i, pl.ds(j * 16, 16)] = zv
        return 0

    lax.fori_loop(0, K, zero_row, 0)
    rbase = s * RZB
    nz = jnp.where(s == NS - 1, (N - (NS - 1) * RZB) // K, RZB // K)

    def zcopy(i, _):
        pltpu.async_copy(mr[0], acc.at[pl.ds(rbase + i * K, K)], sem_s[0])
        return 0

    lax.fori_loop(0, nz, zcopy, 0)

    def zdrain(i, _):
        pltpu.make_async_copy(mr[0], acc.at[pl.ds(rbase, K)], sem_s[0]).wait()
        return 0

    lax.fori_loop(0, nz, zdrain, 0)
    pltpu.async_copy(mr[0], acc.at[di_r[0]], sem_s[0], add=True)
    pltpu.async_copy(mr[1], acc.at[di_r[1]], sem_s[1], add=True)
    plsc.subcore_barrier()

    def step(ch, k):
        ch = jnp.asarray(ch, jnp.int32)
        b = k % 2
        wait_gathers(b, k)
        pltpu.make_async_copy(mr[b], acc.at[di_r[k]], sem_s[b]).wait()

        def row(i, _):
            for j in range(C // 16):
                sl = pl.ds(j * 16, 16)
                mr[b][i, sl] = jnp.maximum(ur[b][i, sl] + vr[b][i, sl], 0.0)
            return 0

        lax.fori_loop(0, K, row, 0)
        pltpu.async_copy(mr[b], acc.at[di_r[k]], sem_s[b], add=True)
        wait_idx((k + 2) % KI)
        issue_gathers(b, (k + 2) % KI)
        issue_idx(ch + PF, (k + PF) % KI)

    def group(g, _):
        for k in range(KI):
            step(g * KI + k, k)
        return 0

    NG = NCHUNK // KI
    lax.fori_loop(0, NG, group, 0)
    for k in range(NCHUNK - NG * KI):
        step(NG * KI + k, k)

    lastb = (NCHUNK - 1) % 2
    pltpu.make_async_copy(mr[1 - lastb], acc.at[di_r[0]],
                          sem_s[1 - lastb]).wait()
    pltpu.make_async_copy(mr[lastb], acc.at[di_r[0]], sem_s[lastb]).wait()
    wait_gathers(0, 2)
    wait_gathers(1, 3)
    for m in range(4):
        wait_idx((NCHUNK + 2 + m) % KI)
    plsc.subcore_barrier()

    @pl.when(s < NS - 1)
    def _():
        pltpu.sync_copy(acc.at[pl.ds(rbase, RZB)],
                        out_hbm.at[c, pl.ds(rbase, RZB)])

    @pl.when(s == NS - 1)
    def _():
        r0 = (NS - 1) * RZB
        pltpu.sync_copy(acc.at[pl.ds(r0, N - (NS - 1) * RZB)],
                        out_hbm.at[c, pl.ds(r0, N - (NS - 1) * RZB)])


@functools.cache
def _get_sc_edges():
    return pl.kernel(
        _sc_edges_body,
        out_type=jax.ShapeDtypeStruct((NC, N, C), jnp.float32),
        mesh=plsc.VectorSubcoreMesh(core_axis_name="c", subcore_axis_name="s",
                                    num_cores=NC, num_subcores=NS),
        scratch_types=(
            [pltpu.VMEM((K, C), jnp.float32)] * 6
            + [pltpu.VMEM((K,), jnp.int32)] * (2 * KI)
            + [pltpu.VMEM_SHARED((N, C), jnp.float32)]
            + [pltpu.SemaphoreType.DMA] * (6 + KI)
        ),
    )


def _prep_block(W_f, b_f, W_h2, b_h2):
    Wfpp = jnp.pad(W_f[:, :3], ((0, 0), (0, C - 3)))
    Wfx = W_f[:, 3:]
    Wh2p = jnp.pad(W_h2, ((0, C - 3), (0, 0)))
    bh2p = jnp.pad(b_h2, (0, C - 3)).reshape(1, C)
    return Wfpp, Wfx, Wh2p, bh2p, b_f.reshape(1, C)


def kernel(h, pos, style, edge_index,
           W_h1_1, b_h1_1, W_h2_1, b_h2_1, W_f_1, b_f_1, W_g1_1, b_g1_1,
           W_g2_1, b_g2_1, W_s_1, b_s_1,
           W_h1_2, b_h1_2, W_h2_2, b_h2_2, W_f_2, b_f_2, W_g1_2, b_g1_2,
           W_g2_2, b_g2_2, W_s_2, b_s_2):
    src = jnp.pad(edge_index[0], (0, PF * K))
    dst = jnp.pad(edge_index[1], (0, PF * K))
    posp = jnp.pad(pos, ((0, 0), (0, C - 3)))

    Wfpp1, Wfx1, Wh2p1, bh2p1, bfr1 = _prep_block(W_f_1, b_f_1, W_h2_1, b_h2_1)
    Wfpp2, Wfx2, Wh2p2, bh2p2, bfr2 = _prep_block(W_f_2, b_f_2, W_h2_2, b_h2_2)
    norm1 = (W_s_1[:C], b_s_1[:C].reshape(1, C), W_s_1[C:], b_s_1[C:].reshape(1, C))
    norm2 = (W_s_2[:C], b_s_2[:C].reshape(1, C), W_s_2[C:], b_s_2[C:].reshape(1, C))
    bh1_1 = b_h1_1.reshape(1, C)
    bh1_2 = b_h1_2.reshape(1, C)
    bg_1 = (b_g1_1.reshape(1, C), b_g2_1.reshape(1, C))
    bg_2 = (b_g1_2.reshape(1, C), b_g2_2.reshape(1, C))

    _sc_edges = _get_sc_edges()
    U1, V1 = _tc_pre(h, posp, W_h1_1, bh1_1, Wh2p1, bh2p1, Wfpp1, Wfx1, bfr1)
    agg1 = _sc_edges(U1, V1, src, dst)
    h1, U2, V2 = _tc_mid(h, agg1, style, posp,
                         W_g1_1, bg_1[0], W_g2_1, bg_1[1],
                         norm1[0], norm1[1], norm1[2], norm1[3],
                         W_h1_2, bh1_2, Wh2p2, bh2p2, Wfpp2, Wfx2, bfr2)
    agg2 = _sc_edges(U2, V2, src, dst)
    h2 = _tc_post(h1, agg2, style,
                  W_g1_2, bg_2[0], W_g2_2, bg_2[1],
                  norm2[0], norm2[1], norm2[2], norm2[3])
    return h2


# speedup vs baseline: 1.0135x; 1.0135x over previous
import functools

import jax
import jax.numpy as jnp
import numpy as np
from jax import lax
from jax.experimental import pallas as pl
from jax.experimental.pallas import tpu as pltpu
from jax.experimental.pallas import tpu_sc as plsc

N = 10000
E = 320000
C = 128

NC = 2
NS = 16
NW = NC * NS
EPW = E // NW
K = 40
NCHUNK = EPW // K
RZB = 640


def _dot_t(x, w):
    return lax.dot_general(x, w, (((1,), (1,)), ((), ())),
                           preferred_element_type=jnp.float32)


def _relu(x):
    return jnp.maximum(x, 0.0)


def _pre_math(h, posp, Wh1, bh1, Wh2p, bh2p, Wfpp, Wfx, bfr):
    z1 = _relu(_dot_t(h, Wh1) + bh1)
    d = jnp.tanh(_dot_t(z1, Wh2p) + bh2p)
    pP = _dot_t(posp, Wfpp)
    U = pP + _dot_t(h, Wfx)
    V = _dot_t(d, Wfpp) - pP + bfr
    return U, V


def _post_math(x, agg, style, Wg1, bg1, Wg2, bg2, Wsg, bsg, Wsb, bsb):
    g1 = _relu(_dot_t(agg, Wg1) + bg1)
    o = _relu(_dot_t(g1, Wg2) + bg2)
    t = x + o
    gam = _dot_t(style, Wsg) + bsg
    bet = _dot_t(style, Wsb) + bsb
    mu = jnp.mean(t, axis=1, keepdims=True)
    var = jnp.mean((t - mu) * (t - mu), axis=1, keepdims=True)
    y = gam * ((t - mu) * lax.rsqrt(var + 1e-5)) + bet
    return jnp.where(y >= 0, y, 0.01 * y)


RT = 1000
GRID = N // RT

_row = lambda r, c=C: pl.BlockSpec((RT, c), lambda i: (i, 0))
_full = lambda a, b: pl.BlockSpec((a, b), lambda i: (0, 0))
_vec = lambda c=C: pl.BlockSpec((1, c), lambda i: (0, 0))


def _tc_pre_body(h, posp, Wh1, bh1, Wh2p, bh2p, Wfpp, Wfx, bfr, U, V):
    u, v = _pre_math(h[...], posp[...], Wh1[...], bh1[...], Wh2p[...],
                     bh2p[...], Wfpp[...], Wfx[...], bfr[...])
    U[...] = u
    V[...] = v


def _tc_mid_body(x, agg2, style, posp,
                 Wg1, bg1, Wg2, bg2, Wsg, bsg, Wsb, bsb,
                 Wh1, bh1, Wh2p, bh2p, Wfpp, Wfx, bfr,
                 Y, U, V):
    agg = agg2[0] + agg2[1]
    y = _post_math(x[...], agg, style[...], Wg1[...], bg1[...], Wg2[...],
                   bg2[...], Wsg[...], bsg[...], Wsb[...], bsb[...])
    Y[...] = y
    u, v = _pre_math(y, posp[...], Wh1[...], bh1[...], Wh2p[...], bh2p[...],
                     Wfpp[...], Wfx[...], bfr[...])
    U[...] = u
    V[...] = v


def _tc_post_body(x, agg2, style,
                  Wg1, bg1, Wg2, bg2, Wsg, bsg, Wsb, bsb, Y):
    agg = agg2[0] + agg2[1]
    Y[...] = _post_math(x[...], agg, style[...], Wg1[...], bg1[...],
                        Wg2[...], bg2[...], Wsg[...], bsg[...], Wsb[...],
                        bsb[...])


_W_SPECS = [_full(C, C), _vec(), _full(C, C), _vec(), _full(C, C), _vec(),
            _full(C, C), _vec()]
_PRE_W_SPECS = [_full(C, C), _vec(), _full(C, C), _vec(), _full(C, C),
                _full(C, C), _vec()]

_tc_pre = pl.pallas_call(
    _tc_pre_body,
    grid=(GRID,),
    in_specs=[_row(RT), _row(RT)] + _PRE_W_SPECS,
    out_specs=[_row(RT), _row(RT)],
    out_shape=[jax.ShapeDtypeStruct((N, C), jnp.float32)] * 2,
)

_agg_spec = pl.BlockSpec((2, RT, C), lambda i: (0, i, 0))

_tc_mid = pl.pallas_call(
    _tc_mid_body,
    grid=(GRID,),
    in_specs=[_row(RT), _agg_spec, _row(RT), _row(RT)] + _W_SPECS + _PRE_W_SPECS,
    out_specs=[_row(RT), _row(RT), _row(RT)],
    out_shape=[jax.ShapeDtypeStruct((N, C), jnp.float32)] * 3,
)

_tc_post = pl.pallas_call(
    _tc_post_body,
    grid=(GRID,),
    in_specs=[_row(RT), _agg_spec, _row(RT)] + _W_SPECS,
    out_specs=_row(RT),
    out_shape=jax.ShapeDtypeStruct((N, C), jnp.float32),
)


KI = 8
PF = 6


def _sc_edges_body(u_hbm, v_hbm, src_hbm, dst_hbm, out_hbm, *scr):
    ur = scr[0:2]
    vr = scr[2:4]
    mr = scr[4:6]
    si_r = scr[6:6 + KI]
    di_r = scr[6 + KI:6 + 2 * KI]
    acc = scr[6 + 2 * KI]
    base_s = 7 + 2 * KI
    sem_u = scr[base_s:base_s + 2]
    sem_v = scr[base_s + 2:base_s + 4]
    sem_s = scr[base_s + 4:base_s + 6]
    sem_i = scr[base_s + 6:base_s + 6 + KI]

    c = lax.axis_index("c")
    s = lax.axis_index("s")
    wid = s * NC + c

    ebase = wid * EPW

    def issue_idx(ch, slot):
        pltpu.async_copy(src_hbm.at[pl.ds(ebase + ch * K, K)], si_r[slot],
                         sem_i[slot])
        pltpu.async_copy(dst_hbm.at[pl.ds(ebase + ch * K, K)], di_r[slot],
                         sem_i[slot])

    def wait_idx(slot):
        pltpu.make_async_copy(src_hbm.at[pl.ds(0, K)], si_r[slot],
                              sem_i[slot]).wait()
        pltpu.make_async_copy(dst_hbm.at[pl.ds(0, K)], di_r[slot],
                              sem_i[slot]).wait()

    def issue_gathers(b, slot):
        pltpu.async_copy(u_hbm.at[si_r[slot]], ur[b], sem_u[b])
        pltpu.async_copy(v_hbm.at[di_r[slot]], vr[b], sem_v[b])

    def wait_gathers(b, slot):
        pltpu.make_async_copy(u_hbm.at[si_r[slot]], ur[b], sem_u[b]).wait()
        pltpu.make_async_copy(v_hbm.at[di_r[slot]], vr[b], sem_v[b]).wait()

    for ch0 in range(PF):
        issue_idx(ch0, ch0)
    wait_idx(0)
    issue_gathers(0, 0)
    wait_idx(1)
    issue_gathers(1, 1)

    zv = jnp.zeros((16,), jnp.float32)

    def zero_row(i, _):
        for j in range(C // 16):
            mr[0][i, pl.ds(j * 16, 16)] = zv
        return 0

    lax.fori_loop(0, K, zero_row, 0)
    rbase = s * RZB
    nz = jnp.where(s == NS - 1, (N - (NS - 1) * RZB) // K, RZB // K)

    def zcopy(i, _):
        pltpu.async_copy(mr[0], acc.at[pl.ds(rbase + i * K, K)], sem_s[0])
        return 0

    lax.fori_loop(0, nz, zcopy, 0)

    def zdrain(i, _):
        pltpu.make_async_copy(mr[0], acc.at[pl.ds(rbase, K)], sem_s[0]).wait()
        return 0

    lax.fori_loop(0, nz, zdrain, 0)
    plsc.subcore_barrier()

    def step(ch, k):
        ch = jnp.asarray(ch, jnp.int32)
        b = k % 2
        wait_gathers(b, k)
        @pl.when(ch >= 2)
        def _():
            pltpu.make_async_copy(mr[b], acc.at[di_r[k]], sem_s[b]).wait()

        def row(i, _):
            for j in range(C // 16):
                sl = pl.ds(j * 16, 16)
                mr[b][i, sl] = jnp.maximum(ur[b][i, sl] + vr[b][i, sl], 0.0)
            return 0

        lax.fori_loop(0, K, row, 0)
        pltpu.async_copy(mr[b], acc.at[di_r[k]], sem_s[b], add=True)

        k2 = (k + 2) % KI
        @pl.when(ch + 2 < NCHUNK)
        def _():
            wait_idx(k2)
            issue_gathers(b, k2)

        kp = (k + PF) % KI
        @pl.when(ch + PF < NCHUNK)
        def _():
            issue_idx(ch + PF, kp)

    def group(g, _):
        for k in range(KI):
            step(g * KI + k, k)
        return 0

    NG = NCHUNK // KI
    lax.fori_loop(0, NG, group, 0)
    for k in range(NCHUNK - NG * KI):
        step(NG * KI + k, k)

    lastb = (NCHUNK - 1) % 2
    pltpu.make_async_copy(mr[1 - lastb], acc.at[di_r[0]],
                          sem_s[1 - lastb]).wait()
    pltpu.make_async_copy(mr[lastb], acc.at[di_r[0]], sem_s[lastb]).wait()
    plsc.subcore_barrier()

    @pl.when(s < NS - 1)
    def _():
        pltpu.sync_copy(acc.at[pl.ds(rbase, RZB)],
                        out_hbm.at[c, pl.ds(rbase, RZB)])

    @pl.when(s == NS - 1)
    def _():
        r0 = (NS - 1) * RZB
        pltpu.sync_copy(acc.at[pl.ds(r0, N - (NS - 1) * RZB)],
                        out_hbm.at[c, pl.ds(r0, N - (NS - 1) * RZB)])


@functools.cache
def _get_sc_edges():
    return pl.kernel(
        _sc_edges_body,
        out_type=jax.ShapeDtypeStruct((NC, N, C), jnp.float32),
        mesh=plsc.VectorSubcoreMesh(core_axis_name="c", subcore_axis_name="s",
                                    num_cores=NC, num_subcores=NS),
        scratch_types=(
            [pltpu.VMEM((K, C), jnp.float32)] * 6
            + [pltpu.VMEM((K,), jnp.int32)] * (2 * KI)
            + [pltpu.VMEM_SHARED((N, C), jnp.float32)]
            + [pltpu.SemaphoreType.DMA] * (6 + KI)
        ),
    )


def _prep_block(W_f, b_f, W_h2, b_h2):
    Wfpp = jnp.pad(W_f[:, :3], ((0, 0), (0, C - 3)))
    Wfx = W_f[:, 3:]
    Wh2p = jnp.pad(W_h2, ((0, C - 3), (0, 0)))
    bh2p = jnp.pad(b_h2, (0, C - 3)).reshape(1, C)
    return Wfpp, Wfx, Wh2p, bh2p, b_f.reshape(1, C)


def kernel(h, pos, style, edge_index,
           W_h1_1, b_h1_1, W_h2_1, b_h2_1, W_f_1, b_f_1, W_g1_1, b_g1_1,
           W_g2_1, b_g2_1, W_s_1, b_s_1,
           W_h1_2, b_h1_2, W_h2_2, b_h2_2, W_f_2, b_f_2, W_g1_2, b_g1_2,
           W_g2_2, b_g2_2, W_s_2, b_s_2):
    src = edge_index[0]
    dst = edge_index[1]
    posp = jnp.pad(pos, ((0, 0), (0, C - 3)))

    Wfpp1, Wfx1, Wh2p1, bh2p1, bfr1 = _prep_block(W_f_1, b_f_1, W_h2_1, b_h2_1)
    Wfpp2, Wfx2, Wh2p2, bh2p2, bfr2 = _prep_block(W_f_2, b_f_2, W_h2_2, b_h2_2)
    norm1 = (W_s_1[:C], b_s_1[:C].reshape(1, C), W_s_1[C:], b_s_1[C:].reshape(1, C))
    norm2 = (W_s_2[:C], b_s_2[:C].reshape(1, C), W_s_2[C:], b_s_2[C:].reshape(1, C))
    bh1_1 = b_h1_1.reshape(1, C)
    bh1_2 = b_h1_2.reshape(1, C)
    bg_1 = (b_g1_1.reshape(1, C), b_g2_1.reshape(1, C))
    bg_2 = (b_g1_2.reshape(1, C), b_g2_2.reshape(1, C))

    _sc_edges = _get_sc_edges()
    U1, V1 = _tc_pre(h, posp, W_h1_1, bh1_1, Wh2p1, bh2p1, Wfpp1, Wfx1, bfr1)
    agg1 = _sc_edges(U1, V1, src, dst)
    h1, U2, V2 = _tc_mid(h, agg1, style, posp,
                         W_g1_1, bg_1[0], W_g2_1, bg_1[1],
                         norm1[0], norm1[1], norm1[2], norm1[3],
                         W_h1_2, bh1_2, Wh2p2, bh2p2, Wfpp2, Wfx2, bfr2)
    agg2 = _sc_edges(U2, V2, src, dst)
    h2 = _tc_post(h1, agg2, style,
                  W_g1_2, bg_2[0], W_g2_2, bg_2[1],
                  norm2[0], norm2[1], norm2[2], norm2[3])
    return h2
